# baseline (speedup 1.0000x reference)
import functools

import jax
import jax.numpy as jnp
from jax import lax
from jax.experimental import pallas as pl
from jax.experimental.pallas import tpu as pltpu
from jax.experimental.pallas import tpu_sc as plsc

H = 1024
FF = 2048
E = 8
K = 2
T = 2048
TK = T * K
TM = 128
R = ((TK + E * (TM - 1)) + TM - 1) // TM * TM
NT = R // TM
NTP = 48
LANES = 128
RT = 256
NB = T // RT
NEG = -1e30

_NC = 2
_NS = 16
_NW = _NC * _NS
_TPW = T // _NW


def _router_body(x_ref, rwt_ref, ltri_ref, logits_ref, routef_ref, routei_ref,
                 cnt_ref, runc_ref):
    i = pl.program_id(0)
    xb = x_ref[...]
    l = jnp.dot(xb, rwt_ref[...], preferred_element_type=jnp.float32)
    logits_ref[...] = l
    lanes = lax.broadcasted_iota(jnp.int32, l.shape, 1)
    lm = jnp.where(lanes < E, l, NEG)
    m1 = jnp.max(lm, axis=1, keepdims=True)
    e1 = jnp.min(jnp.where(lm == m1, lanes, 2 ** 30), axis=1, keepdims=True)
    lm2 = jnp.where(lanes == e1, NEG, lm)
    m2 = jnp.max(lm2, axis=1, keepdims=True)
    e2 = jnp.min(jnp.where(lm2 == m2, lanes, 2 ** 30), axis=1, keepdims=True)
    t = jnp.exp(m2 - m1)
    g1 = 1.0 / (1.0 + t)
    g2 = t / (1.0 + t)
    routef_ref[...] = jnp.where(lanes == 2, g1,
                      jnp.where(lanes == 3, g2, 0.0))
    mA = (lanes == e1).astype(jnp.float32)
    mB = (lanes == e2).astype(jnp.float32)
    csA = jnp.dot(ltri_ref[...], mA, preferred_element_type=jnp.float32)
    csB = jnp.dot(ltri_ref[...], mB, preferred_element_type=jnp.float32)

    @pl.when(i == 0)
    def _():
        runc_ref[...] = jnp.zeros_like(runc_ref)

    runc = runc_ref[...]
    totA = jnp.sum(mA, axis=0, keepdims=True)
    totB = jnp.sum(mB, axis=0, keepdims=True)
    rankA = jnp.sum((csA + runc) * mA, axis=1, keepdims=True)
    rankB = jnp.sum((csB + runc + totA) * mB, axis=1, keepdims=True)
    runc_ref[...] = runc + totA + totB
    cnt_ref[...] = runc_ref[...]
    routei_ref[...] = jnp.where(lanes == 0, e1,
                      jnp.where(lanes == 1, e2,
                      jnp.where(lanes == 2, rankA.astype(jnp.int32),
                      jnp.where(lanes == 3, rankB.astype(jnp.int32), 0))))


def _router(x, rwt, ltri):
    return pl.pallas_call(
        _router_body,
        grid=(NB,),
        in_specs=[pl.BlockSpec((RT, H), lambda i: (i, 0)),
                  pl.BlockSpec((H, LANES), lambda i: (0, 0)),
                  pl.BlockSpec((RT, RT), lambda i: (0, 0))],
        out_specs=[pl.BlockSpec((RT, LANES), lambda i: (i, 0)),
                   pl.BlockSpec((RT, LANES), lambda i: (i, 0)),
                   pl.BlockSpec((RT, LANES), lambda i: (i, 0)),
                   pl.BlockSpec((1, LANES), lambda i: (0, 0))],
        out_shape=[jax.ShapeDtypeStruct((T, LANES), jnp.float32),
                   jax.ShapeDtypeStruct((T, LANES), jnp.float32),
                   jax.ShapeDtypeStruct((T, LANES), jnp.int32),
                   jax.ShapeDtypeStruct((1, LANES), jnp.float32)],
        scratch_shapes=[pltpu.VMEM((1, LANES), jnp.float32)],
        compiler_params=pltpu.CompilerParams(
            dimension_semantics=("arbitrary",)),
    )(x, rwt, ltri)


def _dispatch(sel_a, sel_b, rank_a, rank_b, counts16, x):
    mesh = plsc.VectorSubcoreMesh(core_axis_name="c", subcore_axis_name="s")

    @functools.partial(
        pl.kernel,
        out_type=[jax.ShapeDtypeStruct((R, H), jnp.float32),
                  jax.ShapeDtypeStruct((T,), jnp.int32),
                  jax.ShapeDtypeStruct((T,), jnp.int32),
                  jax.ShapeDtypeStruct((NTP,), jnp.int32)],
        mesh=mesh,
        scratch_types=[pltpu.VMEM((_TPW,), jnp.int32),
                       pltpu.VMEM((_TPW,), jnp.int32),
                       pltpu.VMEM((_TPW,), jnp.int32),
                       pltpu.VMEM((_TPW,), jnp.int32),
                       pltpu.VMEM((16,), jnp.int32),
                       pltpu.VMEM((1, _TPW), jnp.int32),
                       pltpu.VMEM((1, _TPW), jnp.int32),
                       pltpu.VMEM((NTP,), jnp.int32),
                       pltpu.VMEM((_TPW, H), jnp.float32),
                       pltpu.SemaphoreType.DMA],
    )
    def dk(sa_hbm, sb_hbm, ra_hbm, rb_hbm, cnt_hbm, x_hbm,
           xs_hbm, da_hbm, db_hbm, teid_hbm,
           sa_v, sb_v, ra_v, rb_v, cnt_v, da_v, db_v, teid_v,
           rows_v, sem):
        wid = lax.axis_index("s") * _NC + lax.axis_index("c")
        base = wid * _TPW
        pltpu.sync_copy(sa_hbm.at[pl.ds(base, _TPW)], sa_v)
        pltpu.sync_copy(sb_hbm.at[pl.ds(base, _TPW)], sb_v)
        pltpu.sync_copy(ra_hbm.at[pl.ds(base, _TPW)], ra_v)
        pltpu.sync_copy(rb_hbm.at[pl.ds(base, _TPW)], rb_v)
        pltpu.sync_copy(cnt_hbm, cnt_v)
        pltpu.sync_copy(x_hbm.at[pl.ds(base, _TPW)], rows_v)
        cv = cnt_v[...]
        ends = []
        offs = []
        run = jnp.int32(0)
        for e in range(E):
            p = lax.shift_left(lax.shift_right_logical(cv[e] + (TM - 1), 7), 7)
            offs.append(run)
            run = run + p
            ends.append(run)
        for sel_v, rank_v, dst_v in ((sa_v, ra_v, da_v), (sb_v, rb_v, db_v)):
            for i in range(_TPW // 16):
                sv = sel_v[pl.ds(i * 16, 16)]
                d = rank_v[pl.ds(i * 16, 16)]
                for e in range(E):
                    d = d + jnp.where(sv == e, offs[e], 0)
                dst_v[0, pl.ds(i * 16, 16)] = d
        pltpu.sync_copy(da_v.at[0], da_hbm.at[pl.ds(base, _TPW)])
        pltpu.sync_copy(db_v.at[0], db_hbm.at[pl.ds(base, _TPW)])

        @pl.when(wid == 0)
        def _():
            for c in range(NTP // 16):
                tv = (c * 16 + lax.iota(jnp.int32, 16)) * TM
                acc = jnp.zeros((16,), jnp.int32)
                for e in range(E):
                    acc = acc + jnp.where(tv >= ends[e], 1, 0)
                teid_v[pl.ds(c * 16, 16)] = jnp.minimum(acc, E - 1)
            pltpu.sync_copy(teid_v, teid_hbm)

        cpA = pltpu.async_copy(rows_v, xs_hbm.at[da_v.at[0]], sem)
        cpB = pltpu.async_copy(rows_v, xs_hbm.at[db_v.at[0]], sem)
        cpA.wait()
        cpB.wait()

    return dk(sel_a, sel_b, rank_a, rank_b, counts16, x)


def _ffn1_body(eids_ref, xs_ref, w1_ref, w3_ref, h_ref):
    xb = xs_ref[...]
    a = jnp.dot(xb, w1_ref[0], preferred_element_type=jnp.float32)
    b = jnp.dot(xb, w3_ref[0], preferred_element_type=jnp.float32)
    h_ref[...] = a * (1.0 / (1.0 + jnp.exp(-a))) * b


def _ffn1(tile_eid, x_s, w1b, w3b):
    grid_spec = pltpu.PrefetchScalarGridSpec(
        num_scalar_prefetch=1,
        grid=(NT,),
        in_specs=[pl.BlockSpec((TM, H), lambda i, eids: (i, 0)),
                  pl.BlockSpec((1, H, FF), lambda i, eids: (eids[i], 0, 0)),
                  pl.BlockSpec((1, H, FF), lambda i, eids: (eids[i], 0, 0))],
        out_specs=pl.BlockSpec((TM, FF), lambda i, eids: (i, 0)),
    )
    return pl.pallas_call(
        _ffn1_body,
        grid_spec=grid_spec,
        out_shape=jax.ShapeDtypeStruct((R, FF), jnp.float32),
        compiler_params=pltpu.CompilerParams(
            dimension_semantics=("arbitrary",)),
    )(tile_eid, x_s, w1b, w3b)


def _ffn2_body(eids_ref, h_ref, w2_ref, o_ref):
    o_ref[...] = jnp.dot(h_ref[...], w2_ref[0],
                         preferred_element_type=jnp.float32)


def _ffn2(tile_eid, h_s, w2b):
    grid_spec = pltpu.PrefetchScalarGridSpec(
        num_scalar_prefetch=1,
        grid=(NT,),
        in_specs=[pl.BlockSpec((TM, FF), lambda i, eids: (i, 0)),
                  pl.BlockSpec((1, FF, H), lambda i, eids: (eids[i], 0, 0))],
        out_specs=pl.BlockSpec((TM, H), lambda i, eids: (i, 0)),
    )
    return pl.pallas_call(
        _ffn2_body,
        grid_spec=grid_spec,
        out_shape=jax.ShapeDtypeStruct((R, H), jnp.float32),
        compiler_params=pltpu.CompilerParams(
            dimension_semantics=("arbitrary",)),
    )(tile_eid, h_s, w2b)


_CCH = 32


def _combine(dst_a, dst_b, gw_a, gw_b, o_s):
    mesh = plsc.VectorSubcoreMesh(core_axis_name="c", subcore_axis_name="s")

    @functools.partial(
        pl.kernel,
        out_type=jax.ShapeDtypeStruct((T, H), jnp.float32),
        mesh=mesh,
        scratch_types=[pltpu.VMEM((_TPW,), jnp.int32),
                       pltpu.VMEM((_TPW,), jnp.int32),
                       pltpu.VMEM((_TPW, 16), jnp.float32),
                       pltpu.VMEM((_TPW, 16), jnp.float32),
                       pltpu.VMEM((_CCH, H), jnp.float32),
                       pltpu.VMEM((_CCH, H), jnp.float32),
                       pltpu.VMEM((_CCH, H), jnp.float32),
                       pltpu.SemaphoreType.DMA],
    )
    def ck(da_hbm, db_hbm, gwa_hbm, gwb_hbm, os_hbm, y_hbm,
           ia_v, ib_v, gwa_v, gwb_v, rowsa_v, rowsb_v, out_v, sem):
        wid = lax.axis_index("s") * _NC + lax.axis_index("c")
        base = wid * _TPW
        pltpu.sync_copy(da_hbm.at[pl.ds(base, _TPW)], ia_v)
        pltpu.sync_copy(db_hbm.at[pl.ds(base, _TPW)], ib_v)
        pltpu.sync_copy(gwa_hbm.at[pl.ds(base, _TPW)], gwa_v)
        pltpu.sync_copy(gwb_hbm.at[pl.ds(base, _TPW)], gwb_v)
        for c in range(_TPW // _CCH):
            cpA = pltpu.async_copy(
                os_hbm.at[ia_v.at[pl.ds(c * _CCH, _CCH)]], rowsa_v, sem)
            cpB = pltpu.async_copy(
                os_hbm.at[ib_v.at[pl.ds(c * _CCH, _CCH)]], rowsb_v, sem)
            cpA.wait()
            cpB.wait()
            for j in range(_CCH):
                wa = gwa_v[c * _CCH + j, :]
                wb = gwb_v[c * _CCH + j, :]

                def body_v(v, _):
                    ra = rowsa_v[j, pl.ds(v * 16, 16)]
                    rb = rowsb_v[j, pl.ds(v * 16, 16)]
                    out_v[j, pl.ds(v * 16, 16)] = wa * ra + wb * rb
                    return 0

                lax.fori_loop(0, H // 16, body_v, 0)
            pltpu.sync_copy(out_v, y_hbm.at[pl.ds(base + c * _CCH, _CCH)])

    return ck(dst_a, dst_b, gw_a, gw_b, o_s)


def kernel(hidden_states, router_w, w1, w2, w3):
    orig_shape = hidden_states.shape
    x = hidden_states.reshape(T, H).astype(jnp.float32)
    rwt = jnp.zeros((LANES, H), jnp.float32).at[:E].set(
        router_w.astype(jnp.float32)).T
    ltri = jnp.tril(jnp.ones((RT, RT), jnp.float32), -1)

    logits_pad, route_f, route_i, cnt = _router(x, rwt, ltri)
    router_logits = logits_pad[:, :E]

    sel_a = route_i[:, 0]
    sel_b = route_i[:, 1]
    rank_a = route_i[:, 2]
    rank_b = route_i[:, 3]
    counts16 = jnp.pad(cnt[0, :E].astype(jnp.int32), (0, 8))
    gw_a = jnp.broadcast_to(route_f[:, 2:3], (T, 16))
    gw_b = jnp.broadcast_to(route_f[:, 3:4], (T, 16))

    x_s, dst_a, dst_b, teid = _dispatch(sel_a, sel_b, rank_a, rank_b,
                                        counts16, x)
    h_s = _ffn1(teid[:NT], x_s, w1, w3)
    o_s = _ffn2(teid[:NT], h_s, w2)
    y = _combine(dst_a, dst_b, gw_a, gw_b, o_s)

    return y.reshape(orig_shape), router_logits


# speedup vs baseline: 1.1553x; 1.1553x over previous
import functools

import jax
import jax.numpy as jnp
from jax import lax
from jax.experimental import pallas as pl
from jax.experimental.pallas import tpu as pltpu
from jax.experimental.pallas import tpu_sc as plsc

H = 1024
FF = 2048
E = 8
K = 2
T = 2048
TK = T * K
TM = 128
R = ((TK + E * (TM - 1)) + TM - 1) // TM * TM
NT = R // TM
NTP = 48
LANES = 128
RT = 256
NB = T // RT
NEG = -1e30

_NC = 2
_NS = 16
_NW = _NC * _NS
_TPW = T // _NW


def _router_body(x_ref, rwt_ref, ltri_ref, logits_ref, routef_ref, routei_ref,
                 cnt_ref, runc_ref):
    i = pl.program_id(0)
    xb = x_ref[...]
    l = jnp.dot(xb, rwt_ref[...], preferred_element_type=jnp.float32)
    logits_ref[...] = l
    lanes = lax.broadcasted_iota(jnp.int32, l.shape, 1)
    lm = jnp.where(lanes < E, l, NEG)
    m1 = jnp.max(lm, axis=1, keepdims=True)
    e1 = jnp.min(jnp.where(lm == m1, lanes, 2 ** 30), axis=1, keepdims=True)
    lm2 = jnp.where(lanes == e1, NEG, lm)
    m2 = jnp.max(lm2, axis=1, keepdims=True)
    e2 = jnp.min(jnp.where(lm2 == m2, lanes, 2 ** 30), axis=1, keepdims=True)
    t = jnp.exp(m2 - m1)
    g1 = 1.0 / (1.0 + t)
    g2 = t / (1.0 + t)
    routef_ref[...] = jnp.where(lanes == 2, g1,
                      jnp.where(lanes == 3, g2, 0.0))
    mA = (lanes == e1).astype(jnp.float32)
    mB = (lanes == e2).astype(jnp.float32)
    csA = jnp.dot(ltri_ref[...], mA, preferred_element_type=jnp.float32)
    csB = jnp.dot(ltri_ref[...], mB, preferred_element_type=jnp.float32)

    @pl.when(i == 0)
    def _():
        runc_ref[...] = jnp.zeros_like(runc_ref)

    runc = runc_ref[...]
    totA = jnp.sum(mA, axis=0, keepdims=True)
    totB = jnp.sum(mB, axis=0, keepdims=True)
    rankA = jnp.sum((csA + runc) * mA, axis=1, keepdims=True)
    rankB = jnp.sum((csB + runc + totA) * mB, axis=1, keepdims=True)
    runc_ref[...] = runc + totA + totB
    cnt_ref[...] = runc_ref[...]
    routei_ref[...] = jnp.where(lanes == 0, e1,
                      jnp.where(lanes == 1, e2,
                      jnp.where(lanes == 2, rankA.astype(jnp.int32),
                      jnp.where(lanes == 3, rankB.astype(jnp.int32), 0))))


def _router(x, rwt, ltri):
    return pl.pallas_call(
        _router_body,
        grid=(NB,),
        in_specs=[pl.BlockSpec((RT, H), lambda i: (i, 0)),
                  pl.BlockSpec((H, LANES), lambda i: (0, 0)),
                  pl.BlockSpec((RT, RT), lambda i: (0, 0))],
        out_specs=[pl.BlockSpec((RT, LANES), lambda i: (i, 0)),
                   pl.BlockSpec((RT, LANES), lambda i: (i, 0)),
                   pl.BlockSpec((RT, LANES), lambda i: (i, 0)),
                   pl.BlockSpec((1, LANES), lambda i: (0, 0))],
        out_shape=[jax.ShapeDtypeStruct((T, LANES), jnp.float32),
                   jax.ShapeDtypeStruct((T, LANES), jnp.float32),
                   jax.ShapeDtypeStruct((T, LANES), jnp.int32),
                   jax.ShapeDtypeStruct((1, LANES), jnp.float32)],
        scratch_shapes=[pltpu.VMEM((1, LANES), jnp.float32)],
        compiler_params=pltpu.CompilerParams(
            dimension_semantics=("arbitrary",)),
    )(x, rwt, ltri)


def _dispatch(sel_a, sel_b, rank_a, rank_b, counts16, x):
    mesh = plsc.VectorSubcoreMesh(core_axis_name="c", subcore_axis_name="s")

    @functools.partial(
        pl.kernel,
        out_type=[jax.ShapeDtypeStruct((R, H), jnp.float32),
                  jax.ShapeDtypeStruct((T,), jnp.int32),
                  jax.ShapeDtypeStruct((T,), jnp.int32),
                  jax.ShapeDtypeStruct((NTP,), jnp.int32)],
        mesh=mesh,
        scratch_types=[pltpu.VMEM((_TPW,), jnp.int32),
                       pltpu.VMEM((_TPW,), jnp.int32),
                       pltpu.VMEM((_TPW,), jnp.int32),
                       pltpu.VMEM((_TPW,), jnp.int32),
                       pltpu.VMEM((16,), jnp.int32),
                       pltpu.VMEM((1, _TPW), jnp.int32),
                       pltpu.VMEM((1, _TPW), jnp.int32),
                       pltpu.VMEM((NTP,), jnp.int32),
                       pltpu.VMEM((_TPW, H), jnp.float32),
                       pltpu.SemaphoreType.DMA],
    )
    def dk(sa_hbm, sb_hbm, ra_hbm, rb_hbm, cnt_hbm, x_hbm,
           xs_hbm, da_hbm, db_hbm, teid_hbm,
           sa_v, sb_v, ra_v, rb_v, cnt_v, da_v, db_v, teid_v,
           rows_v, sem):
        wid = lax.axis_index("s") * _NC + lax.axis_index("c")
        base = wid * _TPW
        pltpu.sync_copy(sa_hbm.at[pl.ds(base, _TPW)], sa_v)
        pltpu.sync_copy(sb_hbm.at[pl.ds(base, _TPW)], sb_v)
        pltpu.sync_copy(ra_hbm.at[pl.ds(base, _TPW)], ra_v)
        pltpu.sync_copy(rb_hbm.at[pl.ds(base, _TPW)], rb_v)
        pltpu.sync_copy(cnt_hbm, cnt_v)
        pltpu.sync_copy(x_hbm.at[pl.ds(base, _TPW)], rows_v)
        cv = cnt_v[...]
        ends = []
        offs = []
        run = jnp.int32(0)
        for e in range(E):
            p = lax.shift_left(lax.shift_right_logical(cv[e] + (TM - 1), 7), 7)
            offs.append(run)
            run = run + p
            ends.append(run)
        for sel_v, rank_v, dst_v in ((sa_v, ra_v, da_v), (sb_v, rb_v, db_v)):
            for i in range(_TPW // 16):
                sv = sel_v[pl.ds(i * 16, 16)]
                d = rank_v[pl.ds(i * 16, 16)]
                for e in range(E):
                    d = d + jnp.where(sv == e, offs[e], 0)
                dst_v[0, pl.ds(i * 16, 16)] = d
        pltpu.sync_copy(da_v.at[0], da_hbm.at[pl.ds(base, _TPW)])
        pltpu.sync_copy(db_v.at[0], db_hbm.at[pl.ds(base, _TPW)])

        @pl.when(wid == 0)
        def _():
            for c in range(NTP // 16):
                tv = (c * 16 + lax.iota(jnp.int32, 16)) * TM
                acc = jnp.zeros((16,), jnp.int32)
                for e in range(E):
                    acc = acc + jnp.where(tv >= ends[e], 1, 0)
                teid_v[pl.ds(c * 16, 16)] = jnp.minimum(acc, E - 1)
            pltpu.sync_copy(teid_v, teid_hbm)

        cpA = pltpu.async_copy(rows_v, xs_hbm.at[da_v.at[0]], sem)
        cpB = pltpu.async_copy(rows_v, xs_hbm.at[db_v.at[0]], sem)
        cpA.wait()
        cpB.wait()

    return dk(sel_a, sel_b, rank_a, rank_b, counts16, x)


def _ffn_body(eids_ref, xs_ref, w1_ref, w3_ref, w2_ref, o_ref):
    xb = xs_ref[...]
    a = jnp.dot(xb, w1_ref[0], preferred_element_type=jnp.float32)
    b = jnp.dot(xb, w3_ref[0], preferred_element_type=jnp.float32)
    h = a * (1.0 / (1.0 + jnp.exp(-a))) * b
    o_ref[...] = jnp.dot(h, w2_ref[0], preferred_element_type=jnp.float32)


def _ffn(tile_eid, x_s, w1b, w3b, w2b):
    grid_spec = pltpu.PrefetchScalarGridSpec(
        num_scalar_prefetch=1,
        grid=(NT,),
        in_specs=[pl.BlockSpec((TM, H), lambda i, eids: (i, 0)),
                  pl.BlockSpec((1, H, FF), lambda i, eids: (eids[i], 0, 0)),
                  pl.BlockSpec((1, H, FF), lambda i, eids: (eids[i], 0, 0)),
                  pl.BlockSpec((1, FF, H), lambda i, eids: (eids[i], 0, 0))],
        out_specs=pl.BlockSpec((TM, H), lambda i, eids: (i, 0)),
    )
    return pl.pallas_call(
        _ffn_body,
        grid_spec=grid_spec,
        out_shape=jax.ShapeDtypeStruct((R, H), jnp.float32),
        compiler_params=pltpu.CompilerParams(
            dimension_semantics=("arbitrary",)),
    )(tile_eid, x_s, w1b, w3b, w2b)


_CCH = 32


def _combine(dst_a, dst_b, gw_a, gw_b, o_s):
    mesh = plsc.VectorSubcoreMesh(core_axis_name="c", subcore_axis_name="s")

    @functools.partial(
        pl.kernel,
        out_type=jax.ShapeDtypeStruct((T, H), jnp.float32),
        mesh=mesh,
        scratch_types=[pltpu.VMEM((_TPW,), jnp.int32),
                       pltpu.VMEM((_TPW,), jnp.int32),
                       pltpu.VMEM((_TPW, 16), jnp.float32),
                       pltpu.VMEM((_TPW, 16), jnp.float32),
                       pltpu.VMEM((_CCH, H), jnp.float32),
                       pltpu.VMEM((_CCH, H), jnp.float32),
                       pltpu.VMEM((_CCH, H), jnp.float32),
                       pltpu.SemaphoreType.DMA],
    )
    def ck(da_hbm, db_hbm, gwa_hbm, gwb_hbm, os_hbm, y_hbm,
           ia_v, ib_v, gwa_v, gwb_v, rowsa_v, rowsb_v, out_v, sem):
        wid = lax.axis_index("s") * _NC + lax.axis_index("c")
        base = wid * _TPW
        pltpu.sync_copy(da_hbm.at[pl.ds(base, _TPW)], ia_v)
        pltpu.sync_copy(db_hbm.at[pl.ds(base, _TPW)], ib_v)
        pltpu.sync_copy(gwa_hbm.at[pl.ds(base, _TPW)], gwa_v)
        pltpu.sync_copy(gwb_hbm.at[pl.ds(base, _TPW)], gwb_v)
        for c in range(_TPW // _CCH):
            cpA = pltpu.async_copy(
                os_hbm.at[ia_v.at[pl.ds(c * _CCH, _CCH)]], rowsa_v, sem)
            cpB = pltpu.async_copy(
                os_hbm.at[ib_v.at[pl.ds(c * _CCH, _CCH)]], rowsb_v, sem)
            cpA.wait()
            cpB.wait()
            for j in range(_CCH):
                wa = gwa_v[c * _CCH + j, :]
                wb = gwb_v[c * _CCH + j, :]

                def body_v(v, _):
                    ra = rowsa_v[j, pl.ds(v * 16, 16)]
                    rb = rowsb_v[j, pl.ds(v * 16, 16)]
                    out_v[j, pl.ds(v * 16, 16)] = wa * ra + wb * rb
                    return 0

                lax.fori_loop(0, H // 16, body_v, 0)
            pltpu.sync_copy(out_v, y_hbm.at[pl.ds(base + c * _CCH, _CCH)])

    return ck(dst_a, dst_b, gw_a, gw_b, o_s)


def kernel(hidden_states, router_w, w1, w2, w3):
    orig_shape = hidden_states.shape
    x = hidden_states.reshape(T, H).astype(jnp.float32)
    rwt = jnp.zeros((LANES, H), jnp.float32).at[:E].set(
        router_w.astype(jnp.float32)).T
    ltri = jnp.tril(jnp.ones((RT, RT), jnp.float32), -1)

    logits_pad, route_f, route_i, cnt = _router(x, rwt, ltri)
    router_logits = logits_pad[:, :E]

    sel_a = route_i[:, 0]
    sel_b = route_i[:, 1]
    rank_a = route_i[:, 2]
    rank_b = route_i[:, 3]
    counts16 = jnp.pad(cnt[0, :E].astype(jnp.int32), (0, 8))
    gw_a = jnp.broadcast_to(route_f[:, 2:3], (T, 16))
    gw_b = jnp.broadcast_to(route_f[:, 3:4], (T, 16))

    x_s, dst_a, dst_b, teid = _dispatch(sel_a, sel_b, rank_a, rank_b,
                                        counts16, x)
    o_s = _ffn(teid[:NT], x_s, w1, w3, w2)
    y = _combine(dst_a, dst_b, gw_a, gw_b, o_s)

    return y.reshape(orig_shape), router_logits


# speedup vs baseline: 1.1710x; 1.0136x over previous
import functools

import jax
import jax.numpy as jnp
from jax import lax
from jax.experimental import pallas as pl
from jax.experimental.pallas import tpu as pltpu
from jax.experimental.pallas import tpu_sc as plsc

H = 1024
FF = 2048
E = 8
K = 2
T = 2048
TK = T * K
TM = 128
R = ((TK + E * (TM - 1)) + TM - 1) // TM * TM
NT = R // TM
NTP = 48
LANES = 128
RT = 256
NB = T // RT
NEG = -1e30

_NC = 2
_NS = 16
_NW = _NC * _NS
_TPW = T // _NW


def _router_body(x_ref, rwt_ref, ltri_ref, logits_ref, routef_ref, routei_ref,
                 cnt_ref, runc_ref):
    i = pl.program_id(0)
    xb = x_ref[...]
    l = jnp.dot(xb, rwt_ref[...], preferred_element_type=jnp.float32)
    logits_ref[...] = l
    lanes = lax.broadcasted_iota(jnp.int32, l.shape, 1)
    lm = jnp.where(lanes < E, l, NEG)
    m1 = jnp.max(lm, axis=1, keepdims=True)
    e1 = jnp.min(jnp.where(lm == m1, lanes, 2 ** 30), axis=1, keepdims=True)
    lm2 = jnp.where(lanes == e1, NEG, lm)
    m2 = jnp.max(lm2, axis=1, keepdims=True)
    e2 = jnp.min(jnp.where(lm2 == m2, lanes, 2 ** 30), axis=1, keepdims=True)
    t = jnp.exp(m2 - m1)
    g1 = 1.0 / (1.0 + t)
    g2 = t / (1.0 + t)
    routef_ref[...] = jnp.where(lanes == 2, g1,
                      jnp.where(lanes == 3, g2, 0.0))
    mA = (lanes == e1).astype(jnp.float32)
    mB = (lanes == e2).astype(jnp.float32)
    csA = jnp.dot(ltri_ref[...], mA, preferred_element_type=jnp.float32)
    csB = jnp.dot(ltri_ref[...], mB, preferred_element_type=jnp.float32)

    @pl.when(i == 0)
    def _():
        runc_ref[...] = jnp.zeros_like(runc_ref)

    runc = runc_ref[...]
    totA = jnp.sum(mA, axis=0, keepdims=True)
    totB = jnp.sum(mB, axis=0, keepdims=True)
    rankA = jnp.sum((csA + runc) * mA, axis=1, keepdims=True)
    rankB = jnp.sum((csB + runc + totA) * mB, axis=1, keepdims=True)
    runc_ref[...] = runc + totA + totB
    cnt_ref[...] = runc_ref[...]
    routei_ref[...] = jnp.where(lanes == 0, e1,
                      jnp.where(lanes == 1, e2,
                      jnp.where(lanes == 2, rankA.astype(jnp.int32),
                      jnp.where(lanes == 3, rankB.astype(jnp.int32), 0))))


def _router(x, rwt, ltri):
    return pl.pallas_call(
        _router_body,
        grid=(NB,),
        in_specs=[pl.BlockSpec((RT, H), lambda i: (i, 0)),
                  pl.BlockSpec((H, LANES), lambda i: (0, 0)),
                  pl.BlockSpec((RT, RT), lambda i: (0, 0))],
        out_specs=[pl.BlockSpec((RT, LANES), lambda i: (i, 0)),
                   pl.BlockSpec((RT, LANES), lambda i: (i, 0)),
                   pl.BlockSpec((RT, LANES), lambda i: (i, 0)),
                   pl.BlockSpec((1, LANES), lambda i: (0, 0))],
        out_shape=[jax.ShapeDtypeStruct((T, LANES), jnp.float32),
                   jax.ShapeDtypeStruct((T, LANES), jnp.float32),
                   jax.ShapeDtypeStruct((T, LANES), jnp.int32),
                   jax.ShapeDtypeStruct((1, LANES), jnp.float32)],
        scratch_shapes=[pltpu.VMEM((1, LANES), jnp.float32)],
        compiler_params=pltpu.CompilerParams(
            dimension_semantics=("arbitrary",)),
    )(x, rwt, ltri)


def _dispatch(sel_a, sel_b, rank_a, rank_b, counts16, x):
    mesh = plsc.VectorSubcoreMesh(core_axis_name="c", subcore_axis_name="s")

    @functools.partial(
        pl.kernel,
        out_type=[jax.ShapeDtypeStruct((R, H), jnp.float32),
                  jax.ShapeDtypeStruct((T,), jnp.int32),
                  jax.ShapeDtypeStruct((T,), jnp.int32),
                  jax.ShapeDtypeStruct((NTP,), jnp.int32)],
        mesh=mesh,
        scratch_types=[pltpu.VMEM((_TPW,), jnp.int32),
                       pltpu.VMEM((_TPW,), jnp.int32),
                       pltpu.VMEM((_TPW,), jnp.int32),
                       pltpu.VMEM((_TPW,), jnp.int32),
                       pltpu.VMEM((16,), jnp.int32),
                       pltpu.VMEM((1, _TPW), jnp.int32),
                       pltpu.VMEM((1, _TPW), jnp.int32),
                       pltpu.VMEM((NTP,), jnp.int32),
                       pltpu.VMEM((_TPW, H), jnp.float32),
                       pltpu.SemaphoreType.DMA],
    )
    def dk(sa_hbm, sb_hbm, ra_hbm, rb_hbm, cnt_hbm, x_hbm,
           xs_hbm, da_hbm, db_hbm, teid_hbm,
           sa_v, sb_v, ra_v, rb_v, cnt_v, da_v, db_v, teid_v,
           rows_v, sem):
        wid = lax.axis_index("s") * _NC + lax.axis_index("c")
        base = wid * _TPW
        pltpu.sync_copy(sa_hbm.at[pl.ds(base, _TPW)], sa_v)
        pltpu.sync_copy(sb_hbm.at[pl.ds(base, _TPW)], sb_v)
        pltpu.sync_copy(ra_hbm.at[pl.ds(base, _TPW)], ra_v)
        pltpu.sync_copy(rb_hbm.at[pl.ds(base, _TPW)], rb_v)
        pltpu.sync_copy(cnt_hbm, cnt_v)
        cp_rows = pltpu.async_copy(x_hbm.at[pl.ds(base, _TPW)], rows_v, sem)
        cv = cnt_v[...]
        ends = []
        offs = []
        run = jnp.int32(0)
        for e in range(E):
            p = lax.shift_left(lax.shift_right_logical(cv[e] + (TM - 1), 7), 7)
            offs.append(run)
            run = run + p
            ends.append(run)
        for sel_v, rank_v, dst_v in ((sa_v, ra_v, da_v), (sb_v, rb_v, db_v)):
            for i in range(_TPW // 16):
                sv = sel_v[pl.ds(i * 16, 16)]
                d = rank_v[pl.ds(i * 16, 16)]
                for e in range(E):
                    d = d + jnp.where(sv == e, offs[e], 0)
                dst_v[0, pl.ds(i * 16, 16)] = d
        cp_rows.wait()
        cpA = pltpu.async_copy(rows_v, xs_hbm.at[da_v.at[0]], sem)
        cpB = pltpu.async_copy(rows_v, xs_hbm.at[db_v.at[0]], sem)
        pltpu.sync_copy(da_v.at[0], da_hbm.at[pl.ds(base, _TPW)])
        pltpu.sync_copy(db_v.at[0], db_hbm.at[pl.ds(base, _TPW)])

        @pl.when(wid == 0)
        def _():
            for c in range(NTP // 16):
                tv = (c * 16 + lax.iota(jnp.int32, 16)) * TM
                acc = jnp.zeros((16,), jnp.int32)
                for e in range(E):
                    acc = acc + jnp.where(tv >= ends[e], 1, 0)
                teid_v[pl.ds(c * 16, 16)] = jnp.minimum(acc, E - 1)
            pltpu.sync_copy(teid_v, teid_hbm)

        cpA.wait()
        cpB.wait()

    return dk(sel_a, sel_b, rank_a, rank_b, counts16, x)


def _ffn_body(eids_ref, xs_ref, w1_ref, w3_ref, w2_ref, o_ref):
    xb = xs_ref[...]
    a = jnp.dot(xb, w1_ref[0], preferred_element_type=jnp.float32)
    b = jnp.dot(xb, w3_ref[0], preferred_element_type=jnp.float32)
    h = a * (1.0 / (1.0 + jnp.exp(-a))) * b
    o_ref[...] = jnp.dot(h, w2_ref[0], preferred_element_type=jnp.float32)


def _ffn(tile_eid, x_s, w1b, w3b, w2b):
    grid_spec = pltpu.PrefetchScalarGridSpec(
        num_scalar_prefetch=1,
        grid=(NT,),
        in_specs=[pl.BlockSpec((TM, H), lambda i, eids: (i, 0)),
                  pl.BlockSpec((1, H, FF), lambda i, eids: (eids[i], 0, 0)),
                  pl.BlockSpec((1, H, FF), lambda i, eids: (eids[i], 0, 0)),
                  pl.BlockSpec((1, FF, H), lambda i, eids: (eids[i], 0, 0))],
        out_specs=pl.BlockSpec((TM, H), lambda i, eids: (i, 0)),
    )
    return pl.pallas_call(
        _ffn_body,
        grid_spec=grid_spec,
        out_shape=jax.ShapeDtypeStruct((R, H), jnp.float32),
        compiler_params=pltpu.CompilerParams(
            dimension_semantics=("arbitrary",)),
    )(tile_eid, x_s, w1b, w3b, w2b)


_CCH = 16
_NCH = _TPW // _CCH


def _combine(dst_a, dst_b, gw_a, gw_b, o_s):
    mesh = plsc.VectorSubcoreMesh(core_axis_name="c", subcore_axis_name="s")

    @functools.partial(
        pl.kernel,
        out_type=jax.ShapeDtypeStruct((T, H), jnp.float32),
        mesh=mesh,
        scratch_types=[pltpu.VMEM((_TPW,), jnp.int32),
                       pltpu.VMEM((_TPW,), jnp.int32),
                       pltpu.VMEM((_TPW, 16), jnp.float32),
                       pltpu.VMEM((_TPW, 16), jnp.float32),
                       pltpu.VMEM((2, _CCH, H), jnp.float32),
                       pltpu.VMEM((2, _CCH, H), jnp.float32),
                       pltpu.VMEM((_CCH, H), jnp.float32),
                       pltpu.SemaphoreType.DMA,
                       pltpu.SemaphoreType.DMA],
    )
    def ck(da_hbm, db_hbm, gwa_hbm, gwb_hbm, os_hbm, y_hbm,
           ia_v, ib_v, gwa_v, gwb_v, rowsa_v, rowsb_v, out_v, sem0, sem1):
        wid = lax.axis_index("s") * _NC + lax.axis_index("c")
        base = wid * _TPW
        pltpu.sync_copy(da_hbm.at[pl.ds(base, _TPW)], ia_v)
        pltpu.sync_copy(db_hbm.at[pl.ds(base, _TPW)], ib_v)
        pltpu.sync_copy(gwa_hbm.at[pl.ds(base, _TPW)], gwa_v)
        pltpu.sync_copy(gwb_hbm.at[pl.ds(base, _TPW)], gwb_v)
        sems = (sem0, sem1)

        def fire(c):
            buf = c % 2
            s = sems[buf]
            return (pltpu.async_copy(
                        os_hbm.at[ia_v.at[pl.ds(c * _CCH, _CCH)]],
                        rowsa_v.at[buf], s),
                    pltpu.async_copy(
                        os_hbm.at[ib_v.at[pl.ds(c * _CCH, _CCH)]],
                        rowsb_v.at[buf], s))

        inflight = fire(0)
        for c in range(_NCH):
            buf = c % 2
            inflight[0].wait()
            inflight[1].wait()
            if c + 1 < _NCH:
                inflight = fire(c + 1)
            for j in range(_CCH):
                wa = gwa_v[c * _CCH + j, :]
                wb = gwb_v[c * _CCH + j, :]

                def body_v(v, _):
                    ra = rowsa_v[buf, j, pl.ds(v * 16, 16)]
                    rb = rowsb_v[buf, j, pl.ds(v * 16, 16)]
                    out_v[j, pl.ds(v * 16, 16)] = wa * ra + wb * rb
                    return 0

                lax.fori_loop(0, H // 16, body_v, 0)
            pltpu.sync_copy(out_v, y_hbm.at[pl.ds(base + c * _CCH, _CCH)])

    return ck(dst_a, dst_b, gw_a, gw_b, o_s)


def kernel(hidden_states, router_w, w1, w2, w3):
    orig_shape = hidden_states.shape
    x = hidden_states.reshape(T, H).astype(jnp.float32)
    rwt = jnp.zeros((LANES, H), jnp.float32).at[:E].set(
        router_w.astype(jnp.float32)).T
    ltri = jnp.tril(jnp.ones((RT, RT), jnp.float32), -1)

    logits_pad, route_f, route_i, cnt = _router(x, rwt, ltri)
    router_logits = logits_pad[:, :E]

    sel_a = route_i[:, 0]
    sel_b = route_i[:, 1]
    rank_a = route_i[:, 2]
    rank_b = route_i[:, 3]
    counts16 = jnp.pad(cnt[0, :E].astype(jnp.int32), (0, 8))
    gw_a = jnp.broadcast_to(route_f[:, 2:3], (T, 16))
    gw_b = jnp.broadcast_to(route_f[:, 3:4], (T, 16))

    x_s, dst_a, dst_b, teid = _dispatch(sel_a, sel_b, rank_a, rank_b,
                                        counts16, x)
    o_s = _ffn(teid[:NT], x_s, w1, w3, w2)
    y = _combine(dst_a, dst_b, gw_a, gw_b, o_s)

    return y.reshape(orig_shape), router_logits
